# baseline (speedup 1.0000x reference)
import functools

import jax
import jax.numpy as jnp
from jax import lax
from jax.experimental import pallas as pl
from jax.experimental.pallas import tpu as pltpu
from jax.experimental.pallas import tpu_sc as plsc

N = 10000
NPAD = 10240
E = 320000
DFEAT = 128
D1 = 16
D2 = 32

NC = 2
NS = 16
LANES = 16
CHUNK = 128
NCH_TILE = 80
NBUF = 8
TOT_CH = NC * NS * NCH_TILE
REAL_CH = E // CHUNK
STRIPE = NPAD // NS
JUNK_ROW = N + 16

P1 = N * D1 // 128
P1F = NPAD * D1 // 128
P2 = N * D2 // 128
P2F = NPAD * D2 // 128


@functools.cache
def _mesh():
    return plsc.VectorSubcoreMesh(core_axis_name="c", subcore_axis_name="s",
                                  num_cores=NC, num_subcores=NS)


def _zero_fill(buf, nrows, width):
    @pl.loop(0, nrows)
    def _(i):
        for k in range(width // LANES):
            buf[i, pl.ds(k * LANES, LANES)] = jnp.zeros((LANES,), jnp.float32)


def _sc_degree(dst_hbm, out_hbm, acc, dst_v, ones_v, zbuf):
    c = lax.axis_index("c")
    s = lax.axis_index("s")
    wid = c * NS + s
    _zero_fill(zbuf, STRIPE, D1)

    @pl.loop(0, CHUNK)
    def _(i):
        ones_v[i, :] = jnp.ones((LANES,), jnp.float32)

    pltpu.sync_copy(zbuf, acc.at[pl.ds(s * STRIPE, STRIPE)])
    pltpu.sync_copy(dst_hbm.at[pl.ds(wid * NCH_TILE, NCH_TILE)], dst_v)
    plsc.subcore_barrier()

    @pl.loop(0, NCH_TILE)
    def _(j):
        pltpu.sync_copy(ones_v, acc.at[dst_v.at[j]], add=True)

    plsc.subcore_barrier()
    pltpu.sync_copy(acc.at[pl.ds(s * STRIPE, STRIPE)],
                    out_hbm.at[c, pl.ds(s * STRIPE, STRIPE)])


@functools.cache
def _degree_call():
    return pl.kernel(
        _sc_degree,
        out_type=jax.ShapeDtypeStruct((NC, NPAD, D1), jnp.float32),
        mesh=_mesh(),
        scratch_types=[
            pltpu.VMEM_SHARED((NPAD, D1), jnp.float32),
            pltpu.VMEM((NCH_TILE, CHUNK), jnp.int32),
            pltpu.VMEM((CHUNK, D1), jnp.float32),
            pltpu.VMEM((STRIPE, D1), jnp.float32),
        ],
    )


def _sc_agg(D, g_hbm, src_hbm, dst_hbm, out_hbm,
            acc, g_sp, src_v, dst_v, rows, zbuf, *sems):
    gsem = sems[:NBUF]
    ssem = sems[NBUF:]
    c = lax.axis_index("c")
    s = lax.axis_index("s")
    wid = c * NS + s
    _zero_fill(zbuf, STRIPE, D)
    pltpu.sync_copy(zbuf, acc.at[pl.ds(s * STRIPE, STRIPE)])
    pltpu.sync_copy(g_hbm.at[pl.ds(s * (N // NS), N // NS)],
                    g_sp.at[pl.ds(s * (N // NS), N // NS)])
    pltpu.sync_copy(src_hbm.at[pl.ds(wid * NCH_TILE, NCH_TILE)], src_v)
    pltpu.sync_copy(dst_hbm.at[pl.ds(wid * NCH_TILE, NCH_TILE)], dst_v)
    plsc.subcore_barrier()
    for b in range(NBUF):
        pltpu.async_copy(g_sp.at[src_v.at[b]], rows.at[b], gsem[b])

    @pl.loop(0, NCH_TILE // NBUF)
    def _(gi):
        for b in range(NBUF):
            j = gi * NBUF + b
            pltpu.make_async_copy(g_sp.at[src_v.at[j]], rows.at[b],
                                  gsem[b]).wait()
            pltpu.async_copy(rows.at[b], acc.at[dst_v.at[j]], ssem[b],
                             add=True)
            bp = (b - 1) % NBUF
            jp = j - 1

            @pl.when(jnp.logical_and(jp >= 0, jp + NBUF < NCH_TILE))
            def _():
                pltpu.make_async_copy(rows.at[bp], acc.at[dst_v.at[0]],
                                      ssem[bp]).wait()
                pltpu.async_copy(g_sp.at[src_v.at[jp + NBUF]], rows.at[bp],
                                 gsem[bp])

    for b in range(NBUF):
        pltpu.make_async_copy(rows.at[b], acc.at[dst_v.at[0]],
                              ssem[b]).wait()
    plsc.subcore_barrier()
    pltpu.sync_copy(acc.at[pl.ds(s * STRIPE, STRIPE)],
                    out_hbm.at[c, pl.ds(s * STRIPE, STRIPE)])


@functools.cache
def _make_agg_call(D):
    return pl.kernel(
        functools.partial(_sc_agg, D),
        out_type=jax.ShapeDtypeStruct((NC, NPAD, D), jnp.float32),
        mesh=_mesh(),
        compiler_params=pltpu.CompilerParams(use_tc_tiling_on_sc=False),
        scratch_types=[
            pltpu.VMEM_SHARED((NPAD, D), jnp.float32),
            pltpu.VMEM_SHARED((N, D), jnp.float32),
            pltpu.VMEM((NCH_TILE, CHUNK), jnp.int32),
            pltpu.VMEM((NCH_TILE, CHUNK), jnp.int32),
            pltpu.VMEM((NBUF, CHUNK, D), jnp.float32),
            pltpu.VMEM((STRIPE, D), jnp.float32),
        ] + [pltpu.SemaphoreType.DMA] * (2 * NBUF),
    )


def _pack128(v, k):
    a = 128 // k
    m = v.shape[0]
    v3 = v.reshape(m // a, a, k)
    return jnp.concatenate([v3[:, i, :] for i in range(a)], axis=1)


def _unpack128(v, k):
    a = 128 // k
    r = v.shape[0]
    parts = [v[:, i * k:(i + 1) * k] for i in range(a)]
    return jnp.stack(parts, axis=1).reshape(r * a, k)


def _tc_prep(e_ref, src_ref, dst_ref):
    ei = e_ref[...]
    src_ref[0:REAL_CH] = ei[0].reshape(REAL_CH, CHUNK)
    dst_ref[0:REAL_CH] = ei[1].reshape(REAL_CH, CHUNK)
    src_ref[REAL_CH:TOT_CH] = jnp.zeros((TOT_CH - REAL_CH, CHUNK), jnp.int32)
    dst_ref[REAL_CH:TOT_CH] = jnp.full((TOT_CH - REAL_CH, CHUNK), JUNK_ROW,
                                       jnp.int32)


def _tc_h1(x_ref, w1_ref, hp_ref):
    h = jnp.dot(x_ref[...], w1_ref[...], preferred_element_type=jnp.float32)
    hp_ref[...] = _pack128(h, D1)


def _tc_g1(degp_ref, hp_ref, g1p_ref, dis16_ref, dis32_ref):
    degp = degp_ref[...]
    degsum = degp[0:P1F] + degp[P1F:2 * P1F] + 1.0
    dis16 = lax.rsqrt(degsum)
    g1p_ref[...] = hp_ref[...] * dis16[0:P1]
    dis16_ref[...] = dis16[0:P1]
    dis_std = _unpack128(dis16, D1)[:, 0:1]
    dis32_ref[...] = _pack128(
        jnp.broadcast_to(dis_std, (NPAD, D2)), D2)[0:P2]


def _tc_mid(s1p_ref, g1p_ref, dis16_ref, dis32_ref, b1_ref, w2_ref, g2p_ref):
    s1p = s1p_ref[...]
    s1sum = s1p[0:P1F] + s1p[P1F:2 * P1F]
    a1 = (s1sum[0:P1] + g1p_ref[...]) * dis16_ref[...] \
        + jnp.tile(b1_ref[...], 8)[None, :]
    z = _unpack128(jnp.maximum(a1, 0.0), D1)
    h2 = jnp.dot(z, w2_ref[...], preferred_element_type=jnp.float32)
    g2 = h2 * _unpack128(dis32_ref[...], D2)
    g2p_ref[...] = _pack128(g2, D2)


def _tc_post(s2p_ref, g2p_ref, dis32_ref, b2_ref, o_ref):
    s2p = s2p_ref[...]
    s2sum = s2p[0:P2F] + s2p[P2F:2 * P2F]
    a2p = (s2sum[0:P2] + g2p_ref[...]) * dis32_ref[...] \
        + jnp.tile(b2_ref[...], 4)[None, :]
    a2 = _unpack128(a2p, D2)
    m = jnp.max(a2, axis=1, keepdims=True)
    lse = jnp.log(jnp.sum(jnp.exp(a2 - m), axis=1, keepdims=True)) + m
    o_ref[...] = a2 - lse


_tc_prep_call = pl.pallas_call(
    _tc_prep,
    out_shape=[jax.ShapeDtypeStruct((TOT_CH, CHUNK), jnp.int32),
               jax.ShapeDtypeStruct((TOT_CH, CHUNK), jnp.int32)],
)

_tc_h1_call = pl.pallas_call(
    _tc_h1,
    out_shape=jax.ShapeDtypeStruct((P1, 128), jnp.float32),
)

_tc_g1_call = pl.pallas_call(
    _tc_g1,
    out_shape=[jax.ShapeDtypeStruct((P1, 128), jnp.float32),
               jax.ShapeDtypeStruct((P1, 128), jnp.float32),
               jax.ShapeDtypeStruct((P2, 128), jnp.float32)],
)

_tc_mid_call = pl.pallas_call(
    _tc_mid,
    out_shape=jax.ShapeDtypeStruct((P2, 128), jnp.float32),
)

_tc_post_call = pl.pallas_call(
    _tc_post,
    out_shape=jax.ShapeDtypeStruct((N, D2), jnp.float32),
)


def kernel(x, edge_index, W1, b1, W2, b2):
    src_p, dst_p = _tc_prep_call(edge_index.astype(jnp.int32))

    h_p = _tc_h1_call(x, W1)
    deg_parts = _degree_call()(dst_p)
    degp = deg_parts.reshape(2 * P1F, 128)
    g1p, dis16, dis32 = _tc_g1_call(degp, h_p)

    s1 = _make_agg_call(D1)(g1p.reshape(N, D1), src_p, dst_p)
    g2p = _tc_mid_call(s1.reshape(2 * P1F, 128), g1p, dis16, dis32, b1, W2)

    s2 = _make_agg_call(D2)(g2p.reshape(N, D2), src_p, dst_p)
    return _tc_post_call(s2.reshape(2 * P2F, 128), g2p, dis32, b2)


# speedup vs baseline: 1.3757x; 1.3757x over previous
import functools

import jax
import jax.numpy as jnp
from jax import lax
from jax.experimental import pallas as pl
from jax.experimental.pallas import tpu as pltpu
from jax.experimental.pallas import tpu_sc as plsc

N = 10000
NPAD = 10240
E = 320000
DFEAT = 128
D1 = 16
D2 = 32

NC = 2
NS = 16
LANES = 16
CHUNK = 128
NCH_TILE = 80
NBUF = 8
TOT_CH = NC * NS * NCH_TILE
REAL_CH = E // CHUNK
STRIPE = NPAD // NS
JUNK_ROW = N + 16

P1 = N * D1 // 128
P1F = NPAD * D1 // 128
P2 = N * D2 // 128
P2F = NPAD * D2 // 128


@functools.cache
def _mesh():
    return plsc.VectorSubcoreMesh(core_axis_name="c", subcore_axis_name="s",
                                  num_cores=NC, num_subcores=NS)


def _zero_fill(buf, nrows, width):
    @pl.loop(0, nrows)
    def _(i):
        for k in range(width // LANES):
            buf[i, pl.ds(k * LANES, LANES)] = jnp.zeros((LANES,), jnp.float32)


def _sc_degree(dst_hbm, out_hbm, acc, dst_v, ones_v, zbuf):
    c = lax.axis_index("c")
    s = lax.axis_index("s")
    wid = c * NS + s
    _zero_fill(zbuf, STRIPE, D1)

    @pl.loop(0, CHUNK)
    def _(i):
        ones_v[i, :] = jnp.ones((LANES,), jnp.float32)

    pltpu.sync_copy(zbuf, acc.at[pl.ds(s * STRIPE, STRIPE)])
    pltpu.sync_copy(dst_hbm.at[pl.ds(wid * NCH_TILE, NCH_TILE)], dst_v)
    plsc.subcore_barrier()

    @pl.loop(0, NCH_TILE)
    def _(j):
        pltpu.sync_copy(ones_v, acc.at[dst_v.at[j]], add=True)

    plsc.subcore_barrier()
    pltpu.sync_copy(acc.at[pl.ds(s * STRIPE, STRIPE)],
                    out_hbm.at[c, pl.ds(s * STRIPE, STRIPE)])


@functools.cache
def _degree_call():
    return pl.kernel(
        _sc_degree,
        out_type=jax.ShapeDtypeStruct((NC, NPAD, D1), jnp.float32),
        mesh=_mesh(),
        compiler_params=pltpu.CompilerParams(use_tc_tiling_on_sc=False),
        scratch_types=[
            pltpu.VMEM_SHARED((NPAD, D1), jnp.float32),
            pltpu.VMEM((NCH_TILE, CHUNK), jnp.int32),
            pltpu.VMEM((CHUNK, D1), jnp.float32),
            pltpu.VMEM((STRIPE, D1), jnp.float32),
        ],
    )


def _sc_agg(D, g_hbm, src_hbm, dst_hbm, out_hbm,
            acc, g_sp, src_v, dst_v, rows, zbuf, *sems):
    gsem = sems[:NBUF]
    ssem = sems[NBUF:]
    c = lax.axis_index("c")
    s = lax.axis_index("s")
    wid = c * NS + s
    _zero_fill(zbuf, STRIPE, D)
    pltpu.sync_copy(zbuf, acc.at[pl.ds(s * STRIPE, STRIPE)])
    pltpu.sync_copy(g_hbm.at[pl.ds(s * (N // NS), N // NS)],
                    g_sp.at[pl.ds(s * (N // NS), N // NS)])
    pltpu.sync_copy(src_hbm.at[pl.ds(wid * NCH_TILE, NCH_TILE)], src_v)
    pltpu.sync_copy(dst_hbm.at[pl.ds(wid * NCH_TILE, NCH_TILE)], dst_v)
    plsc.subcore_barrier()
    for b in range(NBUF):
        pltpu.async_copy(g_sp.at[src_v.at[b]], rows.at[b], gsem[b])

    @pl.loop(0, NCH_TILE // NBUF)
    def _(gi):
        for b in range(NBUF):
            j = gi * NBUF + b
            pltpu.make_async_copy(g_sp.at[src_v.at[j]], rows.at[b],
                                  gsem[b]).wait()
            pltpu.async_copy(rows.at[b], acc.at[dst_v.at[j]], ssem[b],
                             add=True)
            bp = (b - 1) % NBUF
            jp = j - 1

            @pl.when(jnp.logical_and(jp >= 0, jp + NBUF < NCH_TILE))
            def _():
                pltpu.make_async_copy(rows.at[bp], acc.at[dst_v.at[0]],
                                      ssem[bp]).wait()
                pltpu.async_copy(g_sp.at[src_v.at[jp + NBUF]], rows.at[bp],
                                 gsem[bp])

    for b in range(NBUF):
        pltpu.make_async_copy(rows.at[b], acc.at[dst_v.at[0]],
                              ssem[b]).wait()
    plsc.subcore_barrier()
    pltpu.sync_copy(acc.at[pl.ds(s * STRIPE, STRIPE)],
                    out_hbm.at[c, pl.ds(s * STRIPE, STRIPE)])


@functools.cache
def _make_agg_call(D):
    return pl.kernel(
        functools.partial(_sc_agg, D),
        out_type=jax.ShapeDtypeStruct((NC, NPAD, D), jnp.float32),
        mesh=_mesh(),
        compiler_params=pltpu.CompilerParams(use_tc_tiling_on_sc=False),
        scratch_types=[
            pltpu.VMEM_SHARED((NPAD, D), jnp.float32),
            pltpu.VMEM_SHARED((N, D), jnp.float32),
            pltpu.VMEM((NCH_TILE, CHUNK), jnp.int32),
            pltpu.VMEM((NCH_TILE, CHUNK), jnp.int32),
            pltpu.VMEM((NBUF, CHUNK, D), jnp.float32),
            pltpu.VMEM((STRIPE, D), jnp.float32),
        ] + [pltpu.SemaphoreType.DMA] * (2 * NBUF),
    )


def _pack128(v, k):
    a = 128 // k
    m = v.shape[0]
    v3 = v.reshape(m // a, a, k)
    return jnp.concatenate([v3[:, i, :] for i in range(a)], axis=1)


def _unpack128(v, k):
    a = 128 // k
    r = v.shape[0]
    parts = [v[:, i * k:(i + 1) * k] for i in range(a)]
    return jnp.stack(parts, axis=1).reshape(r * a, k)


def _tc_prep(e_ref, src_ref, dst_ref):
    ei = e_ref[...]
    src_ref[0:REAL_CH] = ei[0].reshape(REAL_CH, CHUNK)
    dst_ref[0:REAL_CH] = ei[1].reshape(REAL_CH, CHUNK)
    src_ref[REAL_CH:TOT_CH] = jnp.zeros((TOT_CH - REAL_CH, CHUNK), jnp.int32)
    dst_ref[REAL_CH:TOT_CH] = jnp.full((TOT_CH - REAL_CH, CHUNK), JUNK_ROW,
                                       jnp.int32)


def _tc_h1(x4_ref, w1_ref, hp_ref):
    w1 = w1_ref[...]
    hp_ref[...] = jnp.concatenate(
        [jnp.dot(x4_ref[:, a, :], w1, preferred_element_type=jnp.float32)
         for a in range(8)], axis=1)


def _tc_g1(degp_ref, hp_ref, g1p_ref, dis16_ref, dis32_ref):
    degp = degp_ref[...]
    degsum = degp[0:P1F] + degp[P1F:2 * P1F] + 1.0
    dis16 = lax.rsqrt(degsum)
    g1p_ref[...] = hp_ref[...] * dis16[0:P1]
    dis16_ref[...] = dis16[0:P1]
    li = lax.broadcasted_iota(jnp.int32, (128, 128), 0)
    lo = lax.broadcasted_iota(jnp.int32, (128, 128), 1)
    sel = 16 * (lo // D2)
    m_even = jnp.where(li == sel, 1.0, 0.0)
    m_odd = jnp.where(li == sel + 64, 1.0, 0.0)
    d_even = jnp.dot(dis16, m_even, preferred_element_type=jnp.float32)
    d_odd = jnp.dot(dis16, m_odd, preferred_element_type=jnp.float32)
    dis32 = jnp.concatenate([d_even, d_odd], axis=1).reshape(2 * P1F, 128)
    dis32_ref[...] = dis32[0:P2]


def _tc_mid(s1p_ref, g1p_ref, dis16_ref, dis32_ref, b1_ref, w2_ref, g2p_ref):
    s1p = s1p_ref[...]
    s1sum = s1p[0:P1F] + s1p[P1F:2 * P1F]
    a1 = (s1sum[0:P1] + g1p_ref[...]) * dis16_ref[...] \
        + jnp.tile(b1_ref[...], 8)[None, :]
    z_p = jnp.maximum(a1, 0.0)
    w2t = jnp.tile(w2_ref[...], (8, 8))
    ri = lax.broadcasted_iota(jnp.int32, (128, 2 * 128), 0)
    ci = lax.broadcasted_iota(jnp.int32, (128, 2 * 128), 1)
    bd = jnp.where(ri // D1 == ci // D2, w2t, 0.0)
    h2p = jnp.dot(z_p, bd, preferred_element_type=jnp.float32)
    g2p_ref[...] = h2p.reshape(P2, 128) * dis32_ref[...]


def _tc_post(s2p_ref, g2p_ref, dis32_ref, b2_ref, o_ref):
    s2p = s2p_ref[...]
    s2sum = s2p[0:P2F] + s2p[P2F:2 * P2F]
    a2p = (s2sum[0:P2] + g2p_ref[...]) * dis32_ref[...] \
        + jnp.tile(b2_ref[...], 4)[None, :]
    a2 = _unpack128(a2p, D2)
    m = jnp.max(a2, axis=1, keepdims=True)
    lse = jnp.log(jnp.sum(jnp.exp(a2 - m), axis=1, keepdims=True)) + m
    o_ref[...] = a2 - lse


_tc_prep_call = pl.pallas_call(
    _tc_prep,
    out_shape=[jax.ShapeDtypeStruct((TOT_CH, CHUNK), jnp.int32),
               jax.ShapeDtypeStruct((TOT_CH, CHUNK), jnp.int32)],
)

_tc_h1_call = pl.pallas_call(
    _tc_h1,
    out_shape=jax.ShapeDtypeStruct((P1, 128), jnp.float32),
)

_tc_g1_call = pl.pallas_call(
    _tc_g1,
    out_shape=[jax.ShapeDtypeStruct((P1, 128), jnp.float32),
               jax.ShapeDtypeStruct((P1, 128), jnp.float32),
               jax.ShapeDtypeStruct((P2, 128), jnp.float32)],
)

_tc_mid_call = pl.pallas_call(
    _tc_mid,
    out_shape=jax.ShapeDtypeStruct((P2, 128), jnp.float32),
)

_tc_post_call = pl.pallas_call(
    _tc_post,
    out_shape=jax.ShapeDtypeStruct((N, D2), jnp.float32),
)


def kernel(x, edge_index, W1, b1, W2, b2):
    src_p, dst_p = _tc_prep_call(edge_index.astype(jnp.int32))

    h_p = _tc_h1_call(x.reshape(P1, 8, 128), W1)
    deg_parts = _degree_call()(dst_p)
    degp = deg_parts.reshape(2 * P1F, 128)
    g1p, dis16, dis32 = _tc_g1_call(degp, h_p)

    s1 = _make_agg_call(D1)(g1p.reshape(N, D1), src_p, dst_p)
    g2p = _tc_mid_call(s1.reshape(2 * P1F, 128), g1p, dis16, dis32, b1, W2)

    s2 = _make_agg_call(D2)(g2p.reshape(N, D2), src_p, dst_p)
    return _tc_post_call(s2.reshape(2 * P2F, 128), g2p, dis32, b2)
